# baseline (speedup 1.0000x reference)
import functools
import jax
import jax.numpy as jnp
from jax import lax
from jax.experimental import pallas as pl
from jax.experimental.pallas import tpu as pltpu
from jax.experimental.pallas import tpu_sc as plsc

NC = 2
NS = 16
NW = NC * NS
CHUNK = 256


def _mm2_body(x_ref, wa_ref, wb_ref, oa_ref, ob_ref):
    x = x_ref[...]
    n = x_ref.shape[0]
    oa_ref[:n] = jnp.dot(x, wa_ref[...], preferred_element_type=jnp.float32)
    if oa_ref.shape[0] > n:
        oa_ref[n:] = jnp.zeros((oa_ref.shape[0] - n, oa_ref.shape[1]),
                               jnp.float32)
    ob_ref[...] = jnp.dot(x, wb_ref[...], preferred_element_type=jnp.float32)


def _mid_body(xroot_ref, parts_ref, b_ref, wrel_ref, wroot_ref, hr_ref, hroot_ref):
    n = xroot_ref.shape[0]
    h = xroot_ref[...] + parts_ref[0, :n] + parts_ref[1, :n] + b_ref[...]
    h = jnp.maximum(h, 0.0)
    hr_ref[:n] = jnp.dot(h, wrel_ref[...], preferred_element_type=jnp.float32)
    if hr_ref.shape[0] > n:
        hr_ref[n:] = jnp.zeros((hr_ref.shape[0] - n, hr_ref.shape[1]),
                               jnp.float32)
    hroot_ref[...] = jnp.dot(h, wroot_ref[...], preferred_element_type=jnp.float32)


def _final_body(hroot_ref, parts_ref, b_ref, o_ref):
    n = hroot_ref.shape[0]
    z = hroot_ref[...] + parts_ref[0, :n] + parts_ref[1, :n] + b_ref[...]
    m = jnp.max(z, axis=1, keepdims=True)
    lse = m + jnp.log(jnp.sum(jnp.exp(z - m), axis=1, keepdims=True))
    o_ref[...] = z - lse


def _sc_scatter_body(ept, acc_rows, n_nodes, d,
                     table, eir, out, idx_s, idx_d, rowbuf0, tab_sh,
                     acc, sem0):
    c = lax.axis_index("c")
    s = lax.axis_index("s")
    wid = s * NC + c
    pltpu.sync_copy(eir.at[0].at[wid], idx_s)
    pltpu.sync_copy(eir.at[1].at[wid], idx_d)
    trows = table.shape[0] // NS
    pltpu.sync_copy(table.at[pl.ds(s * trows, trows)],
                    tab_sh.at[pl.ds(s * trows, trows)])
    zvec = jnp.zeros((16,), jnp.float32)

    def zfill(i, carry):
        rowbuf0[i // (d // 16), pl.ds((i % (d // 16)) * 16, 16)] = zvec
        return carry

    lax.fori_loop(0, 128 * d // 16, zfill, 0)
    zrows = acc_rows // NS
    for k in range(zrows // 128):
        pltpu.sync_copy(rowbuf0.at[pl.ds(0, 128)],
                        acc.at[pl.ds(s * zrows + k * 128, 128)])
    plsc.subcore_barrier()

    n_ch = -(-ept // CHUNK)

    def chunk(i, carry):
        pltpu.async_copy(tab_sh.at[idx_s.at[i]], rowbuf0, sem0).wait()
        pltpu.sync_copy(rowbuf0, acc.at[idx_d.at[i]], add=True)
        return carry

    lax.fori_loop(0, n_ch, chunk, 0)
    plsc.subcore_barrier()
    pltpu.sync_copy(acc.at[pl.ds(s * zrows, zrows)],
                    out.at[c].at[pl.ds(s * zrows, zrows)])


def _make_sc_scatter(n_nodes, tpad, d, ept):
    acc_rows = ((n_nodes + NS * CHUNK - 1) // (NS * CHUNK)) * (NS * CHUNK)
    n_ch = -(-ept // CHUNK)
    mesh = plsc.VectorSubcoreMesh(core_axis_name="c", subcore_axis_name="s")
    kern = pl.kernel(
        functools.partial(_sc_scatter_body, ept, acc_rows, n_nodes, d),
        out_type=jax.ShapeDtypeStruct((NC, acc_rows, d), jnp.float32),
        mesh=mesh,
        scratch_types=[
            pltpu.VMEM((n_ch, CHUNK), jnp.int32),
            pltpu.VMEM((n_ch, CHUNK), jnp.int32),
            pltpu.VMEM((CHUNK, d), jnp.float32),
            pltpu.VMEM_SHARED((tpad, d), jnp.float32),
            pltpu.VMEM_SHARED((acc_rows, d), jnp.float32),
            pltpu.SemaphoreType.DMA,
        ],
        compiler_params=pltpu.CompilerParams(use_tc_tiling_on_sc=False),
    )
    return kern, acc_rows


def kernel(x, edge_index, W1_root, W1_rel, b1, W2_root, W2_rel, b2):
    n_nodes, d_in = x.shape
    d_hid = W1_root.shape[1]
    d_out = W2_root.shape[1]
    n_edges = edge_index.shape[1]

    ept = n_edges // NW
    n_ch = -(-ept // CHUNK)

    tpad = ((n_nodes + NS) // NS) * NS
    sc1, acc_rows1 = _make_sc_scatter(n_nodes, tpad, d_hid, ept)
    sc2, acc_rows2 = _make_sc_scatter(n_nodes, tpad, d_out, ept)

    eir = jnp.pad(edge_index.reshape(2, NW, ept),
                  ((0, 0), (0, 0), (0, n_ch * CHUNK - ept)),
                  constant_values=n_nodes).reshape(2, NW, n_ch, CHUNK)

    mm2 = pl.pallas_call(
        _mm2_body,
        out_shape=(jax.ShapeDtypeStruct((tpad, d_hid), jnp.float32),
                   jax.ShapeDtypeStruct((n_nodes, d_hid), jnp.float32)),
    )
    xr1, xroot = mm2(x, W1_rel, W1_root)

    parts1 = sc1(xr1, eir)

    mid = pl.pallas_call(
        _mid_body,
        out_shape=(jax.ShapeDtypeStruct((tpad, d_out), jnp.float32),
                   jax.ShapeDtypeStruct((n_nodes, d_out), jnp.float32)),
    )
    hr, hroot = mid(xroot, parts1, b1.reshape(1, d_hid), W2_rel, W2_root)

    parts2 = sc2(hr, eir)

    final = pl.pallas_call(
        _final_body,
        out_shape=jax.ShapeDtypeStruct((n_nodes, d_out), jnp.float32),
    )
    return final(hroot, parts2, b2.reshape(1, d_out))


# speedup vs baseline: 1.0109x; 1.0109x over previous
import functools
import jax
import jax.numpy as jnp
from jax import lax
from jax.experimental import pallas as pl
from jax.experimental.pallas import tpu as pltpu
from jax.experimental.pallas import tpu_sc as plsc

NC = 2
NS = 16
NW = NC * NS
CHUNK = 128


def _mm2_body(x_ref, wa_ref, wb_ref, oa_ref, ob_ref):
    x = x_ref[...]
    n = x_ref.shape[0]
    oa_ref[:n] = jnp.dot(x, wa_ref[...], preferred_element_type=jnp.float32)
    if oa_ref.shape[0] > n:
        oa_ref[n:] = jnp.zeros((oa_ref.shape[0] - n, oa_ref.shape[1]),
                               jnp.float32)
    ob_ref[...] = jnp.dot(x, wb_ref[...], preferred_element_type=jnp.float32)


def _mid_body(xroot_ref, parts_ref, b_ref, wrel_ref, wroot_ref, hr_ref, hroot_ref):
    n = xroot_ref.shape[0]
    h = xroot_ref[...] + parts_ref[0, :n] + parts_ref[1, :n] + b_ref[...]
    h = jnp.maximum(h, 0.0)
    hr_ref[:n] = jnp.dot(h, wrel_ref[...], preferred_element_type=jnp.float32)
    if hr_ref.shape[0] > n:
        hr_ref[n:] = jnp.zeros((hr_ref.shape[0] - n, hr_ref.shape[1]),
                               jnp.float32)
    hroot_ref[...] = jnp.dot(h, wroot_ref[...], preferred_element_type=jnp.float32)


def _final_body(hroot_ref, parts_ref, b_ref, o_ref):
    n = hroot_ref.shape[0]
    z = hroot_ref[...] + parts_ref[0, :n] + parts_ref[1, :n] + b_ref[...]
    m = jnp.max(z, axis=1, keepdims=True)
    lse = m + jnp.log(jnp.sum(jnp.exp(z - m), axis=1, keepdims=True))
    o_ref[...] = z - lse


def _sc_scatter_body(ept, acc_rows, n_nodes, d,
                     table, eir, out, idx_s, idx_d, rowbuf0, tab_sh,
                     acc, sem0):
    c = lax.axis_index("c")
    s = lax.axis_index("s")
    wid = s * NC + c
    pltpu.sync_copy(eir.at[0].at[wid], idx_s)
    pltpu.sync_copy(eir.at[1].at[wid], idx_d)
    trows = table.shape[0] // NS
    pltpu.sync_copy(table.at[pl.ds(s * trows, trows)],
                    tab_sh.at[pl.ds(s * trows, trows)])
    zvec = jnp.zeros((16,), jnp.float32)

    def zfill(i, carry):
        rowbuf0[i // (d // 16), pl.ds((i % (d // 16)) * 16, 16)] = zvec
        return carry

    lax.fori_loop(0, 128 * d // 16, zfill, 0)
    zrows = acc_rows // NS
    for k in range(zrows // 128):
        pltpu.sync_copy(rowbuf0.at[pl.ds(0, 128)],
                        acc.at[pl.ds(s * zrows + k * 128, 128)])
    plsc.subcore_barrier()

    n_ch = -(-ept // CHUNK)

    def chunk(i, carry):
        pltpu.async_copy(tab_sh.at[idx_s.at[i]], rowbuf0, sem0).wait()
        pltpu.sync_copy(rowbuf0, acc.at[idx_d.at[i]], add=True)
        return carry

    lax.fori_loop(0, n_ch, chunk, 0)
    plsc.subcore_barrier()
    pltpu.sync_copy(acc.at[pl.ds(s * zrows, zrows)],
                    out.at[c].at[pl.ds(s * zrows, zrows)])


def _make_sc_scatter(n_nodes, tpad, d, ept):
    acc_rows = ((n_nodes + NS * CHUNK - 1) // (NS * CHUNK)) * (NS * CHUNK)
    n_ch = -(-ept // CHUNK)
    mesh = plsc.VectorSubcoreMesh(core_axis_name="c", subcore_axis_name="s")
    kern = pl.kernel(
        functools.partial(_sc_scatter_body, ept, acc_rows, n_nodes, d),
        out_type=jax.ShapeDtypeStruct((NC, acc_rows, d), jnp.float32),
        mesh=mesh,
        scratch_types=[
            pltpu.VMEM((n_ch, CHUNK), jnp.int32),
            pltpu.VMEM((n_ch, CHUNK), jnp.int32),
            pltpu.VMEM((CHUNK, d), jnp.float32),
            pltpu.VMEM_SHARED((tpad, d), jnp.float32),
            pltpu.VMEM_SHARED((acc_rows, d), jnp.float32),
            pltpu.SemaphoreType.DMA,
        ],
        compiler_params=pltpu.CompilerParams(use_tc_tiling_on_sc=False),
    )
    return kern, acc_rows


def kernel(x, edge_index, W1_root, W1_rel, b1, W2_root, W2_rel, b2):
    n_nodes, d_in = x.shape
    d_hid = W1_root.shape[1]
    d_out = W2_root.shape[1]
    n_edges = edge_index.shape[1]

    ept = n_edges // NW
    n_ch = -(-ept // CHUNK)

    tpad = ((n_nodes + NS) // NS) * NS
    sc1, acc_rows1 = _make_sc_scatter(n_nodes, tpad, d_hid, ept)
    sc2, acc_rows2 = _make_sc_scatter(n_nodes, tpad, d_out, ept)

    eir = jnp.pad(edge_index.reshape(2, NW, ept),
                  ((0, 0), (0, 0), (0, n_ch * CHUNK - ept)),
                  constant_values=n_nodes).reshape(2, NW, n_ch, CHUNK)

    mm2 = pl.pallas_call(
        _mm2_body,
        out_shape=(jax.ShapeDtypeStruct((tpad, d_hid), jnp.float32),
                   jax.ShapeDtypeStruct((n_nodes, d_hid), jnp.float32)),
    )
    xr1, xroot = mm2(x, W1_rel, W1_root)

    parts1 = sc1(xr1, eir)

    mid = pl.pallas_call(
        _mid_body,
        out_shape=(jax.ShapeDtypeStruct((tpad, d_out), jnp.float32),
                   jax.ShapeDtypeStruct((n_nodes, d_out), jnp.float32)),
    )
    hr, hroot = mid(xroot, parts1, b1.reshape(1, d_hid), W2_rel, W2_root)

    parts2 = sc2(hr, eir)

    final = pl.pallas_call(
        _final_body,
        out_shape=jax.ShapeDtypeStruct((n_nodes, d_out), jnp.float32),
    )
    return final(hroot, parts2, b2.reshape(1, d_out))
